# baseline (speedup 1.0000x reference)
import functools

import jax
import jax.numpy as jnp
from jax import lax
from jax.experimental import pallas as pl
from jax.experimental.pallas import tpu as pltpu
from jax.experimental.pallas import tpu_sc as plsc

NUM_FACTORS = 64
BATCH = 16384
L = 16
NC = 2
NS = 16
NW = NC * NS
B_PER_W = BATCH // NW
GROUPS = B_PER_W // L


def _sc_kernel(uid_hbm, iid_hbm, uf_hbm, if_hbm, w_hbm, bias_hbm, out_hbm,
               uidx_v, iidx_v, urows_v, irows_v, w_v, bias_v, tbuf_v, out_v,
               sem_u, sem_i):
    wid = lax.axis_index("s") * NC + lax.axis_index("c")
    base = wid * B_PER_W

    pltpu.sync_copy(uid_hbm.at[pl.ds(base, B_PER_W)], uidx_v)
    pltpu.sync_copy(iid_hbm.at[pl.ds(base, B_PER_W)], iidx_v)
    pltpu.sync_copy(w_hbm, w_v)
    pltpu.sync_copy(bias_hbm, bias_v)

    cu = pltpu.async_copy(uf_hbm.at[uidx_v], urows_v, sem_u)
    ci = pltpu.async_copy(if_hbm.at[iidx_v], irows_v, sem_i)
    cu.wait()
    ci.wait()

    w0 = w_v[pl.ds(0, L)]
    w1 = w_v[pl.ds(L, L)]
    w2 = w_v[pl.ds(2 * L, L)]
    w3 = w_v[pl.ds(3 * L, L)]
    bias = bias_v[...]
    iota = lax.iota(jnp.int32, L)
    lane_masks = [iota == j for j in range(L)]

    def group_body(g, carry):
        b0 = g * L
        acc = bias
        for j in range(L):
            b = b0 + j
            t = (urows_v[b, pl.ds(0, L)] * irows_v[b, pl.ds(0, L)] * w0
                 + urows_v[b, pl.ds(L, L)] * irows_v[b, pl.ds(L, L)] * w1
                 + urows_v[b, pl.ds(2 * L, L)] * irows_v[b, pl.ds(2 * L, L)] * w2
                 + urows_v[b, pl.ds(3 * L, L)] * irows_v[b, pl.ds(3 * L, L)] * w3)
            acc = jnp.where(lane_masks[j], acc + jnp.sum(t), acc)
        out_v[pl.ds(b0, L)] = acc
        return carry

    lax.fori_loop(0, GROUPS, group_body, 0)

    pltpu.sync_copy(out_v, out_hbm.at[pl.ds(base, B_PER_W)])


@jax.jit
def _run(user_ids, item_ids, user_factors, item_factors, w_vec, bias_splat):
    mesh = plsc.VectorSubcoreMesh(core_axis_name="c", subcore_axis_name="s")
    fn = pl.kernel(
        _sc_kernel,
        mesh=mesh,
        compiler_params=pltpu.CompilerParams(
            needs_layout_passes=False, use_tc_tiling_on_sc=False),
        out_type=jax.ShapeDtypeStruct((BATCH,), jnp.float32),
        scratch_types=[
            pltpu.VMEM((B_PER_W,), jnp.int32),
            pltpu.VMEM((B_PER_W,), jnp.int32),
            pltpu.VMEM((B_PER_W, NUM_FACTORS), jnp.float32),
            pltpu.VMEM((B_PER_W, NUM_FACTORS), jnp.float32),
            pltpu.VMEM((NUM_FACTORS,), jnp.float32),
            pltpu.VMEM((L,), jnp.float32),
            pltpu.VMEM((L * L,), jnp.float32),
            pltpu.VMEM((B_PER_W,), jnp.float32),
            pltpu.SemaphoreType.DMA,
            pltpu.SemaphoreType.DMA,
        ],
    )
    return fn(user_ids, item_ids, user_factors, item_factors, w_vec, bias_splat)


def kernel(user_ids, item_ids, user_factors, item_factors, W, b):
    uid = user_ids.astype(jnp.int32)
    iid = item_ids.astype(jnp.int32)
    w_vec = W.reshape(NUM_FACTORS).astype(jnp.float32)
    bias_splat = jnp.broadcast_to(b.astype(jnp.float32), (L,))
    out = _run(uid, iid, user_factors, item_factors, w_vec, bias_splat)
    return out.reshape(BATCH, 1)
